# Initial kernel scaffold; baseline (speedup 1.0000x reference)
import jax
import jax.numpy as jnp
from jax.experimental import pallas as pl


def kernel(x, table):
    raise NotImplementedError("write your pallas kernel here")


# speedup vs baseline: 12.1762x; 12.1762x over previous
import functools

import jax
import jax.numpy as jnp
from jax import lax
from jax.experimental import pallas as pl
from jax.experimental.pallas import tpu as pltpu
from jax.experimental.pallas import tpu_sc as plsc

_VOCAB = 100000
_EMB = 64
_SEQ = 200
_BATCH = 4096

_NC = 2
_NS = 16
_NW = _NC * _NS
_CB = _BATCH // _NW
_H0 = 112
_H1 = _SEQ - _H0
_NVR = _EMB // 16


def _accumulate(rows, out_v, b, c):
    neg_inf = jnp.full((16,), -jnp.inf, dtype=jnp.float32)
    zero = jnp.zeros((16,), dtype=jnp.float32)
    init = tuple([neg_inf] * _NVR + [zero] * _NVR)

    def make_body(half):
        def body(r, carry):
            out = list(carry)
            for k in range(_NVR):
                v = half[r, pl.ds(16 * k, 16)]
                out[k] = jnp.maximum(out[k], v)
                out[_NVR + k] = out[_NVR + k] + v
            return tuple(out)
        return body

    acc = lax.fori_loop(0, _H0, make_body(rows[b][0]), init, unroll=8)
    acc = lax.fori_loop(0, _H1, make_body(rows[b][1]), acc, unroll=8)
    inv = jnp.float32(1.0 / _SEQ)
    for k in range(_NVR):
        out_v[c, pl.ds(16 * k, 16)] = acc[k]
        out_v[c, pl.ds(_EMB + 16 * k, 16)] = acc[_NVR + k] * inv


def _make_pool_kernel():
    mesh = plsc.VectorSubcoreMesh(core_axis_name="c", subcore_axis_name="s")

    scratch = []
    for _ in range(2):
        scratch += [pltpu.VMEM((_H0,), jnp.int32),
                    pltpu.VMEM((_H1,), jnp.int32),
                    pltpu.VMEM((_H0, _EMB), jnp.float32),
                    pltpu.VMEM((_H1, _EMB), jnp.float32)]
    scratch += [pltpu.VMEM((_CB, 2 * _EMB), jnp.float32)]
    scratch += [pltpu.SemaphoreType.DMA] * 4

    @functools.partial(
        pl.kernel,
        mesh=mesh,
        out_type=jax.ShapeDtypeStruct((_BATCH, 2 * _EMB), jnp.float32),
        scratch_types=scratch,
        compiler_params=pltpu.CompilerParams(use_tc_tiling_on_sc=False),
    )
    def pool(xt_hbm, table_hbm, out_hbm,
             ci00, ci01, r00, r01, ci10, ci11, r10, r11,
             out_v, si0, si1, sg0, sg1):
        cidx = ((ci00, ci01), (ci10, ci11))
        rows = ((r00, r01), (r10, r11))
        isems = (si0, si1)
        gsems = (sg0, sg1)
        wid = lax.axis_index("s") * _NC + lax.axis_index("c")
        base = wid * _CB

        def idx_copies(c, b):
            col = base + c
            return [
                pltpu.make_async_copy(
                    xt_hbm.at[col, pl.ds(0, _H0)], cidx[b][0], isems[b]),
                pltpu.make_async_copy(
                    xt_hbm.at[col, pl.ds(_H0, _H1)], cidx[b][1], isems[b]),
            ]

        def gather_copies(b):
            return [
                pltpu.make_async_copy(
                    table_hbm.at[cidx[b][h]], rows[b][h], gsems[b])
                for h in range(2)
            ]

        def fire_idx(c, b):
            for cp in idx_copies(c, b):
                cp.start()

        def fire_gather(b):
            for cp in idx_copies(jnp.int32(0), b):
                cp.wait()
            for cp in gather_copies(b):
                cp.start()

        fire_idx(jnp.int32(0), 0)
        fire_idx(jnp.int32(1), 1)
        fire_gather(0)

        def outer(i, carry):
            for b in range(2):
                c = i * 2 + b
                for cp in gather_copies(b):
                    cp.wait()
                nxt = c + 2

                @pl.when(nxt < _CB)
                def _():
                    fire_idx(nxt, b)

                @pl.when(c + 1 < _CB)
                def _():
                    fire_gather(1 - b)

                _accumulate(rows, out_v, b, c)
            return carry

        lax.fori_loop(0, _CB // 2, outer, jnp.int32(0))

        pltpu.sync_copy(out_v, out_hbm.at[pl.ds(base, _CB), :])

    return pool


_pool = _make_pool_kernel()


def kernel(x, table):
    xt = x.T
    return _pool(xt, table)
